# baseline (speedup 1.0000x reference)
import dataclasses
import functools

import jax
import jax.numpy as jnp
from jax import lax
from jax.experimental import pallas as pl
from jax.experimental.pallas import tpu as pltpu
from jax.experimental.pallas import tpu_sc as plsc

D = 512
BW = 8
M = 2 * BW + 1
NW = 32
ROWS = D // NW
NSTAGE = 32
WIN = 768
VB = 16
NBUF = 2


def _mlp_body(z_ref, w1_ref, b1_ref, w2_ref, b2_ref, v_ref, h_ref):
    @pl.when(pl.program_id(0) == 0)
    def _():
        pre = jnp.dot(z_ref[...], w1_ref[...],
                      preferred_element_type=jnp.float32,
                      precision=lax.Precision.DEFAULT) + b1_ref[...][None, :]
        h_ref[...] = 0.5 * pre * (1.0 + lax.erf(pre * (2.0 ** -0.5)))
    v_ref[...] = jnp.dot(h_ref[...], w2_ref[...],
                         preferred_element_type=jnp.float32,
                         precision=lax.Precision.DEFAULT) + b2_ref[...]


def _mlp(z, W1, b1, W2, b2):
    B, L = z.shape
    H, N = W2.shape
    nblk = 2
    NB = N // nblk
    return pl.pallas_call(
        _mlp_body,
        grid=(nblk,),
        in_specs=[
            pl.BlockSpec((B, L), lambda i: (0, 0)),
            pl.BlockSpec((L, H), lambda i: (0, 0)),
            pl.BlockSpec((H,), lambda i: (0,)),
            pl.BlockSpec((H, NB), lambda i: (0, i)),
            pl.BlockSpec((1, NB), lambda i: (0, i)),
        ],
        out_specs=pl.BlockSpec((B, NB), lambda i: (0, i)),
        out_shape=jax.ShapeDtypeStruct((B, N), jnp.float32),
        scratch_shapes=[pltpu.VMEM((B, H), jnp.float32)],
    )(z, W1, b1, W2, b2.reshape(1, N))


def _expand_body(v_hbm, out_hbm, v0, v1, ka, kb, si0, si1, soa, sob):
    kbufs = (ka, kb)
    osems = (soa, sob)
    B = v_hbm.shape[0]
    cid = lax.axis_index("core")
    sid = lax.axis_index("subcore")
    wid = sid * 2 + cid
    rb0 = wid * ROWS
    r0 = jnp.clip(rb0 - BW, 0, D - NSTAGE)
    s0 = jnp.minimum((r0 * M) // 128 * 128, D * M - WIN)
    delta = r0 * M - s0

    lane = lax.iota(jnp.int32, 16)
    i_vec = rb0 + lane
    s_vec = jnp.maximum(i_vec - BW, 0)
    hi_vec = jnp.minimum(i_vec + BW, D - 1)
    li17 = (i_vec - r0) * M + delta
    diag_f = 1.0

    for kref in kbufs:
        @pl.loop(0, ROWS)
        def _(r, kref=kref):
            @pl.loop(0, D, step=16)
            def _(c, kref=kref, r=r):
                kref[r, pl.ds(c, 16)] = jnp.zeros((16,), jnp.float32)

    def in_copy(bstart, vbuf, sem):
        return pltpu.make_async_copy(
            v_hbm.at[pl.ds(bstart, VB), pl.ds(s0, WIN)], vbuf, sem)

    def out_copy(kref, b, sem):
        return pltpu.make_async_copy(
            kref, out_hbm.at[b, pl.ds(rb0, ROWS)], sem)

    def fill(kref, vbuf, brow):
        rsp = jnp.full((16,), brow, jnp.int32)
        for m in range(M):
            col = s_vec + m
            valid = col <= hi_vec
            idx2 = (col - r0) * M + (i_vec - jnp.maximum(col - BW, 0)) + delta
            g1 = plsc.load_gather(vbuf, [rsp, li17 + m])
            g2 = plsc.load_gather(vbuf, [rsp, jnp.where(valid, idx2, 0)])
            val = 0.5 * (g1 + g2)
            val = jnp.where(col == i_vec, val + diag_f, val)
            plsc.store_scatter(kref, [lane, col], val, mask=valid)

    def process(bbase, vbuf):
        @pl.loop(0, VB, step=NBUF)
        def _(i):
            b = bbase + i
            for t in range(NBUF):
                @pl.when(b > 0)
                def _(t=t, b=b):
                    out_copy(kbufs[t], b + t, osems[t]).wait()
                fill(kbufs[t], vbuf, i + t)
                out_copy(kbufs[t], b + t, osems[t]).start()

    in_copy(0, v0, si0).start()

    @pl.loop(0, B, step=2 * VB)
    def _(b0):
        in_copy(b0, v0, si0).wait()
        in_copy(b0 + VB, v1, si1).start()
        process(b0, v0)
        in_copy(b0 + VB, v1, si1).wait()

        @pl.when(b0 + 2 * VB < B)
        def _():
            in_copy(b0 + 2 * VB, v0, si0).start()
        process(b0 + VB, v1)

    for t in range(NBUF):
        out_copy(kbufs[t], 0, osems[t]).wait()


def _expand(v):
    B = v.shape[0]
    mesh = plsc.VectorSubcoreMesh(core_axis_name="core",
                                  subcore_axis_name="subcore")
    cp = pltpu.CompilerParams()
    if "needs_layout_passes" in pltpu.CompilerParams.__dataclass_fields__:
        cp = dataclasses.replace(cp, needs_layout_passes=False)
    k = pl.kernel(
        _expand_body,
        out_type=jax.ShapeDtypeStruct((B, D, D), jnp.float32),
        mesh=mesh,
        scratch_types=[
            pltpu.VMEM((VB, WIN), jnp.float32),
            pltpu.VMEM((VB, WIN), jnp.float32),
            pltpu.VMEM((ROWS, D), jnp.float32),
            pltpu.VMEM((ROWS, D), jnp.float32),
            pltpu.SemaphoreType.DMA,
            pltpu.SemaphoreType.DMA,
            pltpu.SemaphoreType.DMA,
            pltpu.SemaphoreType.DMA,
        ],
        compiler_params=cp,
    )
    return k(v)


def kernel(z, W1, b1, W2, b2, mask, ip, mp, jp):
    v = _mlp(z, W1, b1, W2, b2)
    return _expand(v)


# speedup vs baseline: 1.0120x; 1.0120x over previous
import dataclasses
import functools

import jax
import jax.numpy as jnp
from jax import lax
from jax.experimental import pallas as pl
from jax.experimental.pallas import tpu as pltpu
from jax.experimental.pallas import tpu_sc as plsc

D = 512
BW = 8
M = 2 * BW + 1
NW = 32
ROWS = D // NW
NSTAGE = 32
WIN = 768
VB = 16
NBUF = 2


def _mlp_body(z_ref, w1_ref, b1_ref, w2_ref, b2_ref, v_ref, h_ref):
    @pl.when(pl.program_id(0) == 0)
    def _():
        pre = jnp.dot(z_ref[...], w1_ref[...],
                      preferred_element_type=jnp.float32,
                      precision=lax.Precision.DEFAULT) + b1_ref[...][None, :]
        h_ref[...] = 0.5 * pre * (1.0 + lax.erf(pre * (2.0 ** -0.5)))
    v_ref[...] = jnp.dot(h_ref[...], w2_ref[...],
                         preferred_element_type=jnp.float32,
                         precision=lax.Precision.DEFAULT) + b2_ref[...]


def _mlp(z, W1, b1, W2, b2):
    B, L = z.shape
    H, N = W2.shape
    nblk = 2
    NB = N // nblk
    return pl.pallas_call(
        _mlp_body,
        grid=(nblk,),
        in_specs=[
            pl.BlockSpec((B, L), lambda i: (0, 0)),
            pl.BlockSpec((L, H), lambda i: (0, 0)),
            pl.BlockSpec((H,), lambda i: (0,)),
            pl.BlockSpec((H, NB), lambda i: (0, i)),
            pl.BlockSpec((1, NB), lambda i: (0, i)),
        ],
        out_specs=pl.BlockSpec((B, NB), lambda i: (0, i)),
        out_shape=jax.ShapeDtypeStruct((B, N), jnp.float32),
        scratch_shapes=[pltpu.VMEM((B, H), jnp.float32)],
    )(z, W1, b1, W2, b2.reshape(1, N))


def _expand_body(v_hbm, out_hbm, v0, v1, ka, kb, si0, si1, soa, sob):
    kbufs = (ka, kb)
    osems = (soa, sob)
    B = v_hbm.shape[0]
    cid = lax.axis_index("core")
    sid = lax.axis_index("subcore")
    wid = sid * 2 + cid
    rb0 = wid * ROWS
    r0 = jnp.clip(rb0 - BW, 0, D - NSTAGE)
    s0 = jnp.minimum((r0 * M) // 128 * 128, D * M - WIN)
    delta = r0 * M - s0

    lane = lax.iota(jnp.int32, 16)
    i_vec = rb0 + lane
    s_vec = jnp.maximum(i_vec - BW, 0)
    hi_vec = jnp.minimum(i_vec + BW, D - 1)
    li17 = (i_vec - r0) * M + delta
    diag_f = 1.0

    for kref in kbufs:
        @pl.loop(0, ROWS)
        def _(r, kref=kref):
            @pl.loop(0, D, step=16)
            def _(c, kref=kref, r=r):
                kref[r, pl.ds(c, 16)] = jnp.zeros((16,), jnp.float32)

    def in_copy(bstart, vbuf, sem):
        return pltpu.make_async_copy(
            v_hbm.at[pl.ds(bstart, VB), pl.ds(s0, WIN)], vbuf, sem)

    def out_copy(kref, b, sem):
        return pltpu.make_async_copy(
            kref, out_hbm.at[b, pl.ds(rb0, ROWS)], sem)

    def fill(kref, vbuf, brow):
        rsp = jnp.full((16,), brow, jnp.int32)

        @pl.loop(0, M)
        def _(m):
            col = s_vec + m
            valid = col <= hi_vec
            idx2 = (col - r0) * M + (i_vec - jnp.maximum(col - BW, 0)) + delta
            g1 = plsc.load_gather(vbuf, [rsp, li17 + m])
            g2 = plsc.load_gather(vbuf, [rsp, jnp.where(valid, idx2, 0)])
            val = 0.5 * (g1 + g2)
            val = jnp.where(col == i_vec, val + diag_f, val)
            plsc.store_scatter(kref, [lane, col], val, mask=valid)

    def process(bbase, vbuf):
        @pl.loop(0, VB, step=NBUF)
        def _(i):
            b = bbase + i
            for t in range(NBUF):
                @pl.when(b > 0)
                def _(t=t, b=b):
                    out_copy(kbufs[t], b + t, osems[t]).wait()
                fill(kbufs[t], vbuf, i + t)
                out_copy(kbufs[t], b + t, osems[t]).start()

    in_copy(0, v0, si0).start()

    @pl.loop(0, B, step=2 * VB)
    def _(b0):
        in_copy(b0, v0, si0).wait()
        in_copy(b0 + VB, v1, si1).start()
        process(b0, v0)
        in_copy(b0 + VB, v1, si1).wait()

        @pl.when(b0 + 2 * VB < B)
        def _():
            in_copy(b0 + 2 * VB, v0, si0).start()
        process(b0 + VB, v1)

    for t in range(NBUF):
        out_copy(kbufs[t], 0, osems[t]).wait()


def _expand(v):
    B = v.shape[0]
    mesh = plsc.VectorSubcoreMesh(core_axis_name="core",
                                  subcore_axis_name="subcore")
    cp = pltpu.CompilerParams()
    if "needs_layout_passes" in pltpu.CompilerParams.__dataclass_fields__:
        cp = dataclasses.replace(cp, needs_layout_passes=False)
    k = pl.kernel(
        _expand_body,
        out_type=jax.ShapeDtypeStruct((B, D, D), jnp.float32),
        mesh=mesh,
        scratch_types=[
            pltpu.VMEM((VB, WIN), jnp.float32),
            pltpu.VMEM((VB, WIN), jnp.float32),
            pltpu.VMEM((ROWS, D), jnp.float32),
            pltpu.VMEM((ROWS, D), jnp.float32),
            pltpu.SemaphoreType.DMA,
            pltpu.SemaphoreType.DMA,
            pltpu.SemaphoreType.DMA,
            pltpu.SemaphoreType.DMA,
        ],
        compiler_params=cp,
    )
    return k(v)


def kernel(z, W1, b1, W2, b2, mask, ip, mp, jp):
    v = _mlp(z, W1, b1, W2, b2)
    return _expand(v)


# speedup vs baseline: 1.0792x; 1.0664x over previous
import dataclasses
import functools

import jax
import jax.numpy as jnp
from jax import lax
from jax.experimental import pallas as pl
from jax.experimental.pallas import tpu as pltpu
from jax.experimental.pallas import tpu_sc as plsc

D = 512
BW = 8
M = 2 * BW + 1
NW = 32
ROWS = D // NW
NSTAGE = 32
WIN = 768
VB = 32
NBUF = 2


def _mlp_body(z_ref, w1_ref, b1_ref, w2_ref, b2_ref, v_ref, h_ref):
    @pl.when(pl.program_id(0) == 0)
    def _():
        pre = jnp.dot(z_ref[...], w1_ref[...],
                      preferred_element_type=jnp.float32,
                      precision=lax.Precision.DEFAULT) + b1_ref[...][None, :]
        h_ref[...] = 0.5 * pre * (1.0 + lax.erf(pre * (2.0 ** -0.5)))
    v_ref[...] = jnp.dot(h_ref[...], w2_ref[...],
                         preferred_element_type=jnp.float32,
                         precision=lax.Precision.DEFAULT) + b2_ref[...]


def _mlp(z, W1, b1, W2, b2):
    B, L = z.shape
    H, N = W2.shape
    nblk = 2
    NB = N // nblk
    return pl.pallas_call(
        _mlp_body,
        grid=(nblk,),
        in_specs=[
            pl.BlockSpec((B, L), lambda i: (0, 0)),
            pl.BlockSpec((L, H), lambda i: (0, 0)),
            pl.BlockSpec((H,), lambda i: (0,)),
            pl.BlockSpec((H, NB), lambda i: (0, i)),
            pl.BlockSpec((1, NB), lambda i: (0, i)),
        ],
        out_specs=pl.BlockSpec((B, NB), lambda i: (0, i)),
        out_shape=jax.ShapeDtypeStruct((B, N), jnp.float32),
        scratch_shapes=[pltpu.VMEM((B, H), jnp.float32)],
    )(z, W1, b1, W2, b2.reshape(1, N))


def _expand_body(v_hbm, out_hbm, v0, v1, ka, kb, si0, si1, soa, sob):
    kbufs = (ka, kb)
    osems = (soa, sob)
    B = v_hbm.shape[0]
    cid = lax.axis_index("core")
    sid = lax.axis_index("subcore")
    wid = sid * 2 + cid
    rb0 = wid * ROWS
    r0 = jnp.clip(rb0 - BW, 0, D - NSTAGE)
    s0 = jnp.minimum((r0 * M) // 128 * 128, D * M - WIN)
    delta = r0 * M - s0

    lane = lax.iota(jnp.int32, 16)
    i_vec = rb0 + lane
    s_vec = jnp.maximum(i_vec - BW, 0)
    hi_vec = jnp.minimum(i_vec + BW, D - 1)
    li17 = (i_vec - r0) * M + delta
    diag_f = 1.0

    for kref in kbufs:
        @pl.loop(0, ROWS)
        def _(r, kref=kref):
            @pl.loop(0, D, step=64)
            def _(c, kref=kref, r=r):
                for u in range(4):
                    kref[r, pl.ds(c + u * 16, 16)] = jnp.zeros(
                        (16,), jnp.float32)

    def in_copy(bstart, vbuf, sem):
        return pltpu.make_async_copy(
            v_hbm.at[pl.ds(bstart, VB), pl.ds(s0, WIN)], vbuf, sem)

    def out_copy(kref, b, sem):
        return pltpu.make_async_copy(
            kref, out_hbm.at[b, pl.ds(rb0, ROWS)], sem)

    def fill(kref, vbuf, brow):
        rsp = jnp.full((16,), brow, jnp.int32)

        @pl.loop(0, M)
        def _(m):
            col = s_vec + m
            valid = col <= hi_vec
            idx2 = (col - r0) * M + (i_vec - jnp.maximum(col - BW, 0)) + delta
            g1 = plsc.load_gather(vbuf, [rsp, li17 + m])
            g2 = plsc.load_gather(vbuf, [rsp, jnp.where(valid, idx2, 0)])
            val = 0.5 * (g1 + g2)
            val = jnp.where(col == i_vec, val + diag_f, val)
            plsc.store_scatter(kref, [lane, col], val, mask=valid)

    def process(bbase, vbuf):
        @pl.loop(0, VB, step=NBUF)
        def _(i):
            b = bbase + i
            for t in range(NBUF):
                @pl.when(b > 0)
                def _(t=t, b=b):
                    out_copy(kbufs[t], b + t, osems[t]).wait()
                fill(kbufs[t], vbuf, i + t)
                out_copy(kbufs[t], b + t, osems[t]).start()

    in_copy(0, v0, si0).start()

    @pl.loop(0, B, step=2 * VB)
    def _(b0):
        in_copy(b0, v0, si0).wait()
        in_copy(b0 + VB, v1, si1).start()
        process(b0, v0)
        in_copy(b0 + VB, v1, si1).wait()

        @pl.when(b0 + 2 * VB < B)
        def _():
            in_copy(b0 + 2 * VB, v0, si0).start()
        process(b0 + VB, v1)

    for t in range(NBUF):
        out_copy(kbufs[t], 0, osems[t]).wait()


def _expand(v):
    B = v.shape[0]
    mesh = plsc.VectorSubcoreMesh(core_axis_name="core",
                                  subcore_axis_name="subcore")
    cp = pltpu.CompilerParams()
    if "needs_layout_passes" in pltpu.CompilerParams.__dataclass_fields__:
        cp = dataclasses.replace(cp, needs_layout_passes=False)
    k = pl.kernel(
        _expand_body,
        out_type=jax.ShapeDtypeStruct((B, D, D), jnp.float32),
        mesh=mesh,
        scratch_types=[
            pltpu.VMEM((VB, WIN), jnp.float32),
            pltpu.VMEM((VB, WIN), jnp.float32),
            pltpu.VMEM((ROWS, D), jnp.float32),
            pltpu.VMEM((ROWS, D), jnp.float32),
            pltpu.SemaphoreType.DMA,
            pltpu.SemaphoreType.DMA,
            pltpu.SemaphoreType.DMA,
            pltpu.SemaphoreType.DMA,
        ],
        compiler_params=cp,
    )
    return k(v)


def kernel(z, W1, b1, W2, b2, mask, ip, mp, jp):
    v = _mlp(z, W1, b1, W2, b2)
    return _expand(v)


# speedup vs baseline: 1.0993x; 1.0186x over previous
import dataclasses
import functools

import jax
import jax.numpy as jnp
from jax import lax
from jax.experimental import pallas as pl
from jax.experimental.pallas import tpu as pltpu
from jax.experimental.pallas import tpu_sc as plsc

D = 512
BW = 8
M = 2 * BW + 1
NW = 32
ROWS = D // NW
NSTAGE = 32
WIN = 768
VB = 64
NBUF = 2


def _mlp_body(z_ref, w1_ref, b1_ref, w2_ref, b2_ref, v_ref, h_ref):
    @pl.when(pl.program_id(0) == 0)
    def _():
        pre = jnp.dot(z_ref[...], w1_ref[...],
                      preferred_element_type=jnp.float32,
                      precision=lax.Precision.DEFAULT) + b1_ref[...][None, :]
        h_ref[...] = 0.5 * pre * (1.0 + lax.erf(pre * (2.0 ** -0.5)))
    v_ref[...] = jnp.dot(h_ref[...], w2_ref[...],
                         preferred_element_type=jnp.float32,
                         precision=lax.Precision.DEFAULT) + b2_ref[...]


def _mlp(z, W1, b1, W2, b2):
    B, L = z.shape
    H, N = W2.shape
    nblk = 2
    NB = N // nblk
    return pl.pallas_call(
        _mlp_body,
        grid=(nblk,),
        in_specs=[
            pl.BlockSpec((B, L), lambda i: (0, 0)),
            pl.BlockSpec((L, H), lambda i: (0, 0)),
            pl.BlockSpec((H,), lambda i: (0,)),
            pl.BlockSpec((H, NB), lambda i: (0, i)),
            pl.BlockSpec((1, NB), lambda i: (0, i)),
        ],
        out_specs=pl.BlockSpec((B, NB), lambda i: (0, i)),
        out_shape=jax.ShapeDtypeStruct((B, N), jnp.float32),
        scratch_shapes=[pltpu.VMEM((B, H), jnp.float32)],
    )(z, W1, b1, W2, b2.reshape(1, N))


def _expand_body(v_hbm, out_hbm, v0, v1, ka, kb, si0, si1, soa, sob):
    kbufs = (ka, kb)
    osems = (soa, sob)
    B = v_hbm.shape[0]
    cid = lax.axis_index("core")
    sid = lax.axis_index("subcore")
    wid = sid * 2 + cid
    rb0 = wid * ROWS
    r0 = jnp.clip(rb0 - BW, 0, D - NSTAGE)
    s0 = jnp.minimum((r0 * M) // 128 * 128, D * M - WIN)
    delta = r0 * M - s0

    lane = lax.iota(jnp.int32, 16)
    i_vec = rb0 + lane
    s_vec = jnp.maximum(i_vec - BW, 0)
    hi_vec = jnp.minimum(i_vec + BW, D - 1)
    li17 = (i_vec - r0) * M + delta
    diag_f = 1.0

    for kref in kbufs:
        @pl.loop(0, ROWS)
        def _(r, kref=kref):
            @pl.loop(0, D, step=64)
            def _(c, kref=kref, r=r):
                for u in range(4):
                    kref[r, pl.ds(c + u * 16, 16)] = jnp.zeros(
                        (16,), jnp.float32)

    def in_copy(bstart, vbuf, sem):
        return pltpu.make_async_copy(
            v_hbm.at[pl.ds(bstart, VB), pl.ds(s0, WIN)], vbuf, sem)

    def out_copy(kref, b, sem):
        return pltpu.make_async_copy(
            kref, out_hbm.at[b, pl.ds(rb0, ROWS)], sem)

    def fill(kref, vbuf, brow):
        rsp = jnp.full((16,), brow, jnp.int32)

        @pl.loop(0, M)
        def _(m):
            col = s_vec + m
            valid = col <= hi_vec
            idx2 = (col - r0) * M + (i_vec - jnp.maximum(col - BW, 0)) + delta
            g1 = plsc.load_gather(vbuf, [rsp, li17 + m])
            g2 = plsc.load_gather(vbuf, [rsp, jnp.where(valid, idx2, 0)])
            val = 0.5 * (g1 + g2)
            val = jnp.where(col == i_vec, val + diag_f, val)
            plsc.store_scatter(kref, [lane, col], val, mask=valid)

    def process(bbase, vbuf):
        @pl.loop(0, VB, step=NBUF)
        def _(i):
            b = bbase + i
            for t in range(NBUF):
                @pl.when(b > 0)
                def _(t=t, b=b):
                    out_copy(kbufs[t], b + t, osems[t]).wait()
                fill(kbufs[t], vbuf, i + t)
                out_copy(kbufs[t], b + t, osems[t]).start()

    in_copy(0, v0, si0).start()

    @pl.loop(0, B, step=2 * VB)
    def _(b0):
        in_copy(b0, v0, si0).wait()
        in_copy(b0 + VB, v1, si1).start()
        process(b0, v0)
        in_copy(b0 + VB, v1, si1).wait()

        @pl.when(b0 + 2 * VB < B)
        def _():
            in_copy(b0 + 2 * VB, v0, si0).start()
        process(b0 + VB, v1)

    for t in range(NBUF):
        out_copy(kbufs[t], 0, osems[t]).wait()


def _expand(v):
    B = v.shape[0]
    mesh = plsc.VectorSubcoreMesh(core_axis_name="core",
                                  subcore_axis_name="subcore")
    cp = pltpu.CompilerParams()
    if "needs_layout_passes" in pltpu.CompilerParams.__dataclass_fields__:
        cp = dataclasses.replace(cp, needs_layout_passes=False)
    k = pl.kernel(
        _expand_body,
        out_type=jax.ShapeDtypeStruct((B, D, D), jnp.float32),
        mesh=mesh,
        scratch_types=[
            pltpu.VMEM((VB, WIN), jnp.float32),
            pltpu.VMEM((VB, WIN), jnp.float32),
            pltpu.VMEM((ROWS, D), jnp.float32),
            pltpu.VMEM((ROWS, D), jnp.float32),
            pltpu.SemaphoreType.DMA,
            pltpu.SemaphoreType.DMA,
            pltpu.SemaphoreType.DMA,
            pltpu.SemaphoreType.DMA,
        ],
        compiler_params=cp,
    )
    return k(v)


def kernel(z, W1, b1, W2, b2, mask, ip, mp, jp):
    v = _mlp(z, W1, b1, W2, b2)
    return _expand(v)


# speedup vs baseline: 1.1006x; 1.0012x over previous
import dataclasses

import jax
import jax.numpy as jnp
from jax import lax
from jax.experimental import pallas as pl
from jax.experimental.pallas import tpu as pltpu
from jax.experimental.pallas import tpu_sc as plsc

D = 512
BW = 8
M = 2 * BW + 1
NW = 32
ROWS = D // NW
NSTAGE = 32
WIN = 768
VB = 64
NBUF = 2


def _mlp_body(z_ref, w1_ref, b1_ref, w2_ref, b2_ref, v_ref, h_ref):
    @pl.when(pl.program_id(0) == 0)
    def _():
        pre = jnp.dot(z_ref[...], w1_ref[...],
                      preferred_element_type=jnp.float32,
                      precision=lax.Precision.DEFAULT) + b1_ref[...][None, :]
        h_ref[...] = 0.5 * pre * (1.0 + lax.erf(pre * (2.0 ** -0.5)))
    v_ref[...] = jnp.dot(h_ref[...], w2_ref[...],
                         preferred_element_type=jnp.float32,
                         precision=lax.Precision.DEFAULT) + b2_ref[...]


def _mlp(z, W1, b1, W2, b2):
    B, L = z.shape
    H, N = W2.shape
    nblk = 2
    NB = N // nblk
    return pl.pallas_call(
        _mlp_body,
        grid=(nblk,),
        in_specs=[
            pl.BlockSpec((B, L), lambda i: (0, 0)),
            pl.BlockSpec((L, H), lambda i: (0, 0)),
            pl.BlockSpec((H,), lambda i: (0,)),
            pl.BlockSpec((H, NB), lambda i: (0, i)),
            pl.BlockSpec((1, NB), lambda i: (0, i)),
        ],
        out_specs=pl.BlockSpec((B, NB), lambda i: (0, i)),
        out_shape=jax.ShapeDtypeStruct((B, N), jnp.float32),
        scratch_shapes=[pltpu.VMEM((B, H), jnp.float32)],
    )(z, W1, b1, W2, b2.reshape(1, N))


def _expand_body(v_hbm, out_hbm, v0, v1, ka, kb, si0, si1, soa, sob):
    kbufs = (ka, kb)
    osems = (soa, sob)
    B = v_hbm.shape[0]
    cid = lax.axis_index("core")
    sid = lax.axis_index("subcore")
    wid = sid * 2 + cid
    rb0 = wid * ROWS
    r0 = jnp.clip(rb0 - BW, 0, D - NSTAGE)
    s0 = jnp.minimum((r0 * M) // 128 * 128, D * M - WIN)
    delta = r0 * M - s0

    lane = lax.iota(jnp.int32, 16)
    i_vec = rb0 + lane
    s_vec = jnp.maximum(i_vec - BW, 0)
    hi_vec = jnp.minimum(i_vec + BW, D - 1)
    li17 = (i_vec - r0) * M + delta
    diag_f = 1.0

    for kref in kbufs:
        @pl.loop(0, ROWS)
        def _(r, kref=kref):
            @pl.loop(0, D, step=64)
            def _(c, kref=kref, r=r):
                for u in range(4):
                    kref[r, pl.ds(c + u * 16, 16)] = jnp.zeros(
                        (16,), jnp.float32)

    def in_copy(bstart, vbuf, sem):
        return pltpu.make_async_copy(
            v_hbm.at[pl.ds(bstart, VB), pl.ds(s0, WIN)], vbuf, sem)

    def out_copy(kref, b, sem):
        return pltpu.make_async_copy(
            kref, out_hbm.at[b, pl.ds(rb0, ROWS)], sem)

    def fill(kref, vbuf, brow):
        rsp = jnp.full((16,), brow, jnp.int32)

        @pl.loop(0, M)
        def _(m):
            col = s_vec + m
            valid = col <= hi_vec
            idx2 = (col - r0) * M + (i_vec - jnp.maximum(col - BW, 0)) + delta
            g1 = plsc.load_gather(vbuf, [rsp, li17 + m])
            g2 = plsc.load_gather(vbuf, [rsp, jnp.where(valid, idx2, 0)])
            val = 0.5 * (g1 + g2)
            val = jnp.where(col == i_vec, val + diag_f, val)
            plsc.store_scatter(kref, [lane, col], val, mask=valid)

    def process(bbase, vbuf):
        @pl.loop(0, VB, step=NBUF)
        def _(i):
            b = bbase + i
            for t in range(NBUF):
                @pl.when(b > 0)
                def _(t=t, b=b):
                    out_copy(kbufs[t], b + t, osems[t]).wait()
                fill(kbufs[t], vbuf, i + t)
                out_copy(kbufs[t], b + t, osems[t]).start()

    in_copy(0, v0, si0).start()

    @pl.loop(0, B, step=2 * VB)
    def _(b0):
        in_copy(b0, v0, si0).wait()
        in_copy(b0 + VB, v1, si1).start()
        process(b0, v0)
        in_copy(b0 + VB, v1, si1).wait()

        @pl.when(b0 + 2 * VB < B)
        def _():
            in_copy(b0 + 2 * VB, v0, si0).start()
        process(b0 + VB, v1)

    for t in range(NBUF):
        out_copy(kbufs[t], 0, osems[t]).wait()


def _expand(v):
    B = v.shape[0]
    mesh = plsc.VectorSubcoreMesh(core_axis_name="core",
                                  subcore_axis_name="subcore")
    cp = pltpu.CompilerParams()
    if "needs_layout_passes" in pltpu.CompilerParams.__dataclass_fields__:
        cp = dataclasses.replace(cp, needs_layout_passes=False)
    k = pl.kernel(
        _expand_body,
        out_type=jax.ShapeDtypeStruct((B, D, D), jnp.float32),
        mesh=mesh,
        scratch_types=[
            pltpu.VMEM((VB, WIN), jnp.float32),
            pltpu.VMEM((VB, WIN), jnp.float32),
            pltpu.VMEM((ROWS, D), jnp.float32),
            pltpu.VMEM((ROWS, D), jnp.float32),
            pltpu.SemaphoreType.DMA,
            pltpu.SemaphoreType.DMA,
            pltpu.SemaphoreType.DMA,
            pltpu.SemaphoreType.DMA,
        ],
        compiler_params=cp,
    )
    return k(v)


def kernel(z, W1, b1, W2, b2, mask, ip, mp, jp):
    v = _mlp(z, W1, b1, W2, b2)
    return _expand(v)
